# Initial kernel scaffold; baseline (speedup 1.0000x reference)
import jax
import jax.numpy as jnp
from jax.experimental import pallas as pl


def kernel(x, l_0, noise, w1, b1, w2, b2, w34, b34, w_core, b_core, w_heads, b_heads):
    raise NotImplementedError("write your pallas kernel here")


# speedup vs baseline: 1.9936x; 1.9936x over previous
import functools
import math

import jax
import jax.numpy as jnp
from jax.experimental import pallas as pl
from jax.experimental.pallas import tpu as pltpu

_IMG = 64
_PS = 8
_NP = 2
_SCALE = 2
_GLIMPSES = 6
_STD = 0.25


def _step_kernel(
    g_ref, l_ref, noise_ref, h_ref,
    w1_ref, b1_ref, w2_ref, b2_ref, w34_ref, b34_ref,
    wc_ref, bc_ref, whd_ref, bhd_ref,
    h_out, l_out, base_out, logpi_out, logp_out,
    u_ref, gh_ref,
    *, std):
  f32 = jnp.float32
  relu = lambda v: jnp.maximum(v, 0.0)

  hg = w1_ref.shape[1]
  hl = w2_ref.shape[1]
  hid = wc_ref.shape[1]

  what_pre = (jnp.dot(g_ref[...], w1_ref[...], preferred_element_type=f32)
              + b1_ref[...])
  l_in = l_ref[...]
  w2 = w2_ref[...]
  where_pre = (l_in[:, 0:1] * w2[0:1, :] +
               l_in[:, 1:2] * w2[1:2, :] + b2_ref[...])

  u_ref[:, 0:hg] = relu(what_pre)
  u_ref[:, hg:hg + hl] = relu(where_pre)
  g_t = relu(jnp.dot(u_ref[...], w34_ref[...], preferred_element_type=f32)
             + b34_ref[...])

  gh_ref[:, 0:hg + hl] = g_t
  gh_ref[:, hg + hl:hg + hl + hid] = h_ref[...]
  h_t = relu(jnp.dot(gh_ref[...], wc_ref[...], preferred_element_type=f32)
             + bc_ref[...])

  heads = (jnp.dot(h_t, whd_ref[...], preferred_element_type=f32)
           + bhd_ref[...])
  mu = jnp.tanh(heads[:, 0:2])
  baseline = relu(heads[:, 2:3])
  logits = heads[:, 3:]

  l_t = jnp.tanh(mu + noise_ref[...])
  z = (l_t - mu) * (1.0 / std)
  log_norm = -math.log(std) - 0.5 * math.log(2.0 * math.pi)
  log_pi = jnp.sum(-0.5 * z * z + log_norm, axis=1, keepdims=True)

  m = jnp.max(logits, axis=1, keepdims=True)
  shifted = logits - m
  lse = jnp.log(jnp.sum(jnp.exp(shifted), axis=1, keepdims=True))

  h_out[...] = h_t
  l_out[...] = l_t
  base_out[...] = baseline
  logpi_out[...] = log_pi
  logp_out[...] = shifted - lse


def _ram_step(glimpse, l_t, noise_t, h_prev, params, *, std):
  batch = glimpse.shape[0]
  hid = h_prev.shape[1]
  hg = params[0].shape[1]
  hl = params[2].shape[1]
  ncls = params[8].shape[1] - 3

  tb = min(512, ((max(batch, 1) + 7) // 8) * 8)
  bp = ((batch + tb - 1) // tb) * tb
  pad = bp - batch
  data = (glimpse, l_t, noise_t, h_prev)
  if pad:
    data = tuple(jnp.pad(a, ((0, pad), (0, 0))) for a in data)

  grid = (bp // tb,)
  in_specs = [pl.BlockSpec((tb, a.shape[1]), lambda i: (i, 0)) for a in data]
  in_specs += [pl.BlockSpec(p.shape, lambda i: (0, 0)) for p in params]
  out_shapes = [
      jax.ShapeDtypeStruct((bp, hid), jnp.float32),
      jax.ShapeDtypeStruct((bp, 2), jnp.float32),
      jax.ShapeDtypeStruct((bp, 1), jnp.float32),
      jax.ShapeDtypeStruct((bp, 1), jnp.float32),
      jax.ShapeDtypeStruct((bp, ncls), jnp.float32),
  ]
  out_specs = [pl.BlockSpec((tb, s.shape[1]), lambda i: (i, 0))
               for s in out_shapes]

  outs = pl.pallas_call(
      functools.partial(_step_kernel, std=std),
      out_shape=out_shapes,
      grid=grid,
      in_specs=in_specs,
      out_specs=out_specs,
      scratch_shapes=[pltpu.VMEM((tb, hg + hl), jnp.float32),
                      pltpu.VMEM((tb, hg + hl + hid), jnp.float32)],
      compiler_params=pltpu.CompilerParams(
          dimension_semantics=("parallel",)),
  )(*data, *params)
  if pad:
    outs = tuple(o[:batch] for o in outs)
  return outs


def kernel(x, l_0, noise, w1, b1, w2, b2, w34, b34,
           w_core, b_core, w_heads, b_heads):
  batch, chans, height, width = x.shape
  hid = w_core.shape[1]
  params = (w1, b1, w2, b2, w34, b34, w_core, b_core, w_heads, b_heads)

  big = _PS * _SCALE
  pad = big // 2
  xp = jnp.pad(x, ((0, 0), (0, 0), (pad, pad), (pad, pad)))
  img_shape = jnp.array([height, width], jnp.float32)

  def extract16(l):
    coords = (0.5 * ((l + 1.0) * img_shape)).astype(jnp.int32)

    def one(img, fx, fy):
      return jax.lax.dynamic_slice(img, (0, fy, fx), (chans, big, big))

    return jax.vmap(one)(xp, coords[:, 0], coords[:, 1])

  h_t = jnp.zeros((batch, hid), jnp.float32)
  l_t = l_0
  locs, baselines, log_pis = [], [], []
  log_probas = None
  for t in range(_GLIMPSES):
    p16 = extract16(l_t)
    p8 = p16[:, :, _PS // 2:_PS // 2 + _PS, _PS // 2:_PS // 2 + _PS]
    pooled = p16.reshape(batch, chans, _PS, _SCALE, _PS, _SCALE).mean(
        axis=(3, 5))
    glimpse = jnp.concatenate([p8, pooled], axis=1).reshape(batch, -1)
    h_t, l_t, b_t, lp_t, log_probas = _ram_step(
        glimpse, l_t, noise[t], h_t, params, std=_STD)
    locs.append(l_t)
    baselines.append(b_t[:, 0])
    log_pis.append(lp_t[:, 0])
  baselines = jnp.stack(baselines, axis=1)
  log_pis = jnp.stack(log_pis, axis=1)
  return locs, baselines, log_pis, log_probas


# speedup vs baseline: 112.4059x; 56.3841x over previous
import functools
import math

import jax
import jax.numpy as jnp
from jax.experimental import pallas as pl
from jax.experimental.pallas import tpu as pltpu

_IMG = 64
_PS = 8
_SCALE = 2
_BIG = _PS * _SCALE
_GLIMPSES = 6
_STD = 0.25
_TB = 512
_U = 8


def _gather_kernel(coords_ref, x_ref, out_ref, *, tb):
  i = pl.program_id(0)
  iota_r = jax.lax.broadcasted_iota(jnp.int32, (_BIG, _BIG), 0)
  iota_c = jax.lax.broadcasted_iota(jnp.int32, (_BIG, _BIG), 1)
  half = _BIG // 2

  def chunk(k, carry):
    for u in range(_U):
      b = k * _U + u
      g = i * tb + b
      fx = coords_ref[2 * g]
      fy = coords_ref[2 * g + 1]
      ys0 = fy - half
      xs0 = fx - half
      ya = jnp.minimum(jnp.maximum((ys0 >> 3) << 3, 0), _IMG - 32)
      off = ys0 - ya
      start = pl.multiple_of(b * _IMG + ya, 8)
      rows = pltpu.roll(x_ref[pl.ds(start, 32), :], (-off) % 32,
                        axis=0)[0:_BIG, :]
      patch = pltpu.roll(rows, (-xs0) % _IMG, axis=1)[:, 0:_BIG]
      valid = ((iota_r + ys0 >= 0) & (iota_r + ys0 < _IMG) &
               (iota_c + xs0 >= 0) & (iota_c + xs0 < _IMG))
      out_ref[b] = jnp.where(valid, patch, 0.0)
    return carry

  jax.lax.fori_loop(0, tb // _U, chunk, 0)


def _gather16(x2d, coords):
  batch = coords.shape[0]
  tb = min(_TB, batch)
  grid = (batch // tb,)
  return pl.pallas_call(
      functools.partial(_gather_kernel, tb=tb),
      out_shape=jax.ShapeDtypeStruct((batch, _BIG, _BIG), jnp.float32),
      grid=grid,
      in_specs=[pl.BlockSpec(memory_space=pltpu.SMEM),
                pl.BlockSpec((tb * _IMG, _IMG), lambda i: (i, 0))],
      out_specs=pl.BlockSpec((tb, _BIG, _BIG), lambda i: (i, 0, 0)),
      compiler_params=pltpu.CompilerParams(
          dimension_semantics=("parallel",)),
  )(coords.reshape(-1), x2d)


def _step_kernel(
    g_ref, l_ref, noise_ref, h_ref,
    w1_ref, b1_ref, w2_ref, b2_ref, w34_ref, b34_ref,
    wc_ref, bc_ref, whd_ref, bhd_ref,
    h_out, l_out, base_out, logpi_out, logp_out, coords_out,
    u_ref, gh_ref,
    *, std):
  f32 = jnp.float32
  relu = lambda v: jnp.maximum(v, 0.0)

  hg = w1_ref.shape[1]
  hl = w2_ref.shape[1]
  hid = wc_ref.shape[1]

  what_pre = (jnp.dot(g_ref[...], w1_ref[...], preferred_element_type=f32)
              + b1_ref[...])
  l_in = l_ref[...]
  w2 = w2_ref[...]
  where_pre = (l_in[:, 0:1] * w2[0:1, :] +
               l_in[:, 1:2] * w2[1:2, :] + b2_ref[...])

  u_ref[:, 0:hg] = relu(what_pre)
  u_ref[:, hg:hg + hl] = relu(where_pre)
  g_t = relu(jnp.dot(u_ref[...], w34_ref[...], preferred_element_type=f32)
             + b34_ref[...])

  gh_ref[:, 0:hg + hl] = g_t
  gh_ref[:, hg + hl:hg + hl + hid] = h_ref[...]
  h_t = relu(jnp.dot(gh_ref[...], wc_ref[...], preferred_element_type=f32)
             + bc_ref[...])

  heads = (jnp.dot(h_t, whd_ref[...], preferred_element_type=f32)
           + bhd_ref[...])
  mu = jnp.tanh(heads[:, 0:2])
  baseline = relu(heads[:, 2:3])
  logits = heads[:, 3:]

  l_t = jnp.tanh(mu + noise_ref[...])
  z = (l_t - mu) * (1.0 / std)
  log_norm = -math.log(std) - 0.5 * math.log(2.0 * math.pi)
  log_pi = jnp.sum(-0.5 * z * z + log_norm, axis=1, keepdims=True)

  m = jnp.max(logits, axis=1, keepdims=True)
  shifted = logits - m
  lse = jnp.log(jnp.sum(jnp.exp(shifted), axis=1, keepdims=True))

  h_out[...] = h_t
  l_out[...] = l_t
  base_out[...] = baseline
  logpi_out[...] = log_pi
  logp_out[...] = shifted - lse
  coords_out[...] = (0.5 * ((l_t + 1.0) * float(_IMG))).astype(jnp.int32)


def _ram_step(glimpse, l_t, noise_t, h_prev, params, *, std):
  batch = glimpse.shape[0]
  hid = params[6].shape[1]
  hg = params[0].shape[1]
  hl = params[2].shape[1]
  ncls = params[8].shape[1] - 3

  tb = min(_TB, ((max(batch, 1) + 7) // 8) * 8)
  bp = ((batch + tb - 1) // tb) * tb
  pad = bp - batch
  data = (glimpse, l_t, noise_t, h_prev)
  if pad:
    data = tuple(jnp.pad(a, ((0, pad), (0, 0))) for a in data)

  grid = (bp // tb,)
  in_specs = [pl.BlockSpec((tb, a.shape[1]), lambda i: (i, 0)) for a in data]
  in_specs += [pl.BlockSpec(p.shape, lambda i: (0, 0)) for p in params]
  out_shapes = [
      jax.ShapeDtypeStruct((bp, hid), jnp.float32),
      jax.ShapeDtypeStruct((bp, 2), jnp.float32),
      jax.ShapeDtypeStruct((bp, 1), jnp.float32),
      jax.ShapeDtypeStruct((bp, 1), jnp.float32),
      jax.ShapeDtypeStruct((bp, ncls), jnp.float32),
      jax.ShapeDtypeStruct((bp, 2), jnp.int32),
  ]
  out_specs = [pl.BlockSpec((tb, s.shape[1]), lambda i: (i, 0))
               for s in out_shapes]

  outs = pl.pallas_call(
      functools.partial(_step_kernel, std=std),
      out_shape=out_shapes,
      grid=grid,
      in_specs=in_specs,
      out_specs=out_specs,
      scratch_shapes=[pltpu.VMEM((tb, hg + hl), jnp.float32),
                      pltpu.VMEM((tb, hg + hl + hid), jnp.float32)],
      compiler_params=pltpu.CompilerParams(
          dimension_semantics=("parallel",)),
  )(*data, *params)
  if pad:
    outs = tuple(o[:batch] for o in outs)
  return outs


def kernel(x, l_0, noise, w1, b1, w2, b2, w34, b34,
           w_core, b_core, w_heads, b_heads):
  batch, chans, height, width = x.shape
  hid = w_core.shape[1]
  params = (w1, b1, w2, b2, w34, b34, w_core, b_core, w_heads, b_heads)

  x2d = x.reshape(batch * height, width)
  img_shape = jnp.array([height, width], jnp.float32)
  coords = (0.5 * ((l_0 + 1.0) * img_shape)).astype(jnp.int32)

  h_t = jnp.zeros((batch, hid), jnp.float32)
  l_t = l_0
  locs, baselines, log_pis = [], [], []
  log_probas = None
  for t in range(_GLIMPSES):
    p16 = _gather16(x2d, coords).reshape(batch, chans, _BIG, _BIG)
    p8 = p16[:, :, _PS // 2:_PS // 2 + _PS, _PS // 2:_PS // 2 + _PS]
    pooled = p16.reshape(batch, chans, _PS, _SCALE, _PS, _SCALE).mean(
        axis=(3, 5))
    glimpse = jnp.concatenate([p8, pooled], axis=1).reshape(batch, -1)
    h_t, l_t, b_t, lp_t, log_probas, coords = _ram_step(
        glimpse, l_t, noise[t], h_t, params, std=_STD)
    locs.append(l_t)
    baselines.append(b_t[:, 0])
    log_pis.append(lp_t[:, 0])
  baselines = jnp.stack(baselines, axis=1)
  log_pis = jnp.stack(log_pis, axis=1)
  return locs, baselines, log_pis, log_probas


# speedup vs baseline: 133.9269x; 1.1915x over previous
import functools
import math

import jax
import jax.numpy as jnp
from jax.experimental import pallas as pl
from jax.experimental.pallas import tpu as pltpu

_IMG = 64
_PS = 8
_SCALE = 2
_BIG = _PS * _SCALE
_GLIMPSES = 6
_STD = 0.25
_TB = 512
_U = 8


def _make_pool_sel():
  sel = jnp.zeros((8 * _BIG, 8 * _PS), jnp.float32)
  for r in range(_PS):
    for c in range(_PS):
      sel = sel.at[_BIG * r + 2 * c, _PS * r + c].set(1.0)
  return sel


def _fused_kernel(
    x_ref, l0_ref, noise_ref,
    w1_ref, b1_ref, w2_ref, b2_ref, w34_ref, b34_ref,
    wc_ref, bc_ref, whd_ref, bhd_ref, sel_ref,
    loc0, loc1, loc2, loc3, loc4, loc5,
    base_out, logpi_out, logp_out,
    q_ref, cv_ref, cs_ref, u_ref, gh_ref, sem,
    *, tb, std):
  f32 = jnp.float32
  relu = lambda v: jnp.maximum(v, 0.0)
  loc_outs = (loc0, loc1, loc2, loc3, loc4, loc5)

  hg = w1_ref.shape[1]
  hl = w2_ref.shape[1]
  hid = wc_ref.shape[1]
  half = _BIG // 2

  iota_r = jax.lax.broadcasted_iota(jnp.int32, (_BIG, _BIG), 0)
  iota_c = jax.lax.broadcasted_iota(jnp.int32, (_BIG, _BIG), 1)

  def gather_to_q():
    def chunk(k, carry):
      for u in range(_U):
        b = k * _U + u
        fx = cs_ref[b, 0]
        fy = cs_ref[b, 1]
        ys0 = fy - half
        xs0 = fx - half
        ya = jnp.minimum(jnp.maximum((ys0 >> 3) << 3, 0), _IMG - 32)
        off = ys0 - ya
        start = pl.multiple_of(b * _IMG + ya, 8)
        rows = pltpu.roll(x_ref[pl.ds(start, 32), :], (-off) % 32,
                          axis=0)[0:_BIG, :]
        patch = pltpu.roll(rows, (-xs0) % _IMG, axis=1)[:, 0:_BIG]
        valid = ((iota_r + ys0 >= 0) & (iota_r + ys0 < _IMG) &
                 (iota_c + xs0 >= 0) & (iota_c + xs0 < _IMG))
        q_ref[pl.ds(b * _BIG, _BIG), :] = jnp.where(valid, patch, 0.0)
      return carry

    jax.lax.fori_loop(0, tb // _U, chunk, 0)

  def assemble_glimpse():
    rows = [q_ref[i::_BIG, :] for i in range(_BIG)]
    center = jnp.concatenate(
        [rows[half // 2 + r][:, half // 2:half // 2 + _PS]
         for r in range(_PS)], axis=1)
    rsum = [rows[2 * r] + rows[2 * r + 1] for r in range(_PS)]
    usum = jnp.concatenate(
        [v + pltpu.roll(v, _BIG - 1, axis=1) for v in rsum], axis=1)
    pooled = 0.25 * jnp.dot(usum, sel_ref[...],
                            preferred_element_type=f32)
    return jnp.concatenate([center, pooled], axis=1)

  h_t = jnp.zeros((tb, hid), f32)
  l_t = l0_ref[...]
  for t in range(_GLIMPSES):
    cv_ref[...] = (0.5 * ((l_t + 1.0) * float(_IMG))).astype(jnp.int32)
    copy = pltpu.make_async_copy(cv_ref, cs_ref, sem)
    copy.start()
    copy.wait()
    gather_to_q()
    glimpse = assemble_glimpse()

    what_pre = (jnp.dot(glimpse, w1_ref[...], preferred_element_type=f32)
                + b1_ref[...])
    w2 = w2_ref[...]
    where_pre = (l_t[:, 0:1] * w2[0:1, :] +
                 l_t[:, 1:2] * w2[1:2, :] + b2_ref[...])

    u_ref[:, 0:hg] = relu(what_pre)
    u_ref[:, hg:hg + hl] = relu(where_pre)
    g_t = relu(jnp.dot(u_ref[...], w34_ref[...], preferred_element_type=f32)
               + b34_ref[...])

    gh_ref[:, 0:hg + hl] = g_t
    gh_ref[:, hg + hl:hg + hl + hid] = h_t
    h_t = relu(jnp.dot(gh_ref[...], wc_ref[...], preferred_element_type=f32)
               + bc_ref[...])

    heads = (jnp.dot(h_t, whd_ref[...], preferred_element_type=f32)
             + bhd_ref[...])
    mu = jnp.tanh(heads[:, 0:2])
    baseline = relu(heads[:, 2:3])

    l_t = jnp.tanh(mu + noise_ref[t])
    z = (l_t - mu) * (1.0 / std)
    log_norm = -math.log(std) - 0.5 * math.log(2.0 * math.pi)
    log_pi = jnp.sum(-0.5 * z * z + log_norm, axis=1, keepdims=True)

    loc_outs[t][...] = l_t
    base_out[:, t:t + 1] = baseline
    logpi_out[:, t:t + 1] = log_pi
    if t == _GLIMPSES - 1:
      logits = heads[:, 3:]
      m = jnp.max(logits, axis=1, keepdims=True)
      shifted = logits - m
      lse = jnp.log(jnp.sum(jnp.exp(shifted), axis=1, keepdims=True))
      logp_out[...] = shifted - lse


def kernel(x, l_0, noise, w1, b1, w2, b2, w34, b34,
           w_core, b_core, w_heads, b_heads):
  batch, chans, height, width = x.shape
  hid = w_core.shape[1]
  hg = w1.shape[1]
  hl = w2.shape[1]
  ncls = w_heads.shape[1] - 3
  nt = noise.shape[0]

  x2d = x.reshape(batch * height, width)
  tb = min(_TB, batch)
  grid = (batch // tb,)
  sel = _make_pool_sel()

  params = (w1, b1, w2, b2, w34, b34, w_core, b_core, w_heads, b_heads, sel)
  in_specs = [
      pl.BlockSpec((tb * height, width), lambda i: (i, 0)),
      pl.BlockSpec((tb, 2), lambda i: (i, 0)),
      pl.BlockSpec((nt, tb, 2), lambda i: (0, i, 0)),
  ]
  in_specs += [pl.BlockSpec(p.shape, lambda i: (0,) * p.ndim) for p in params]
  out_shapes = (
      [jax.ShapeDtypeStruct((batch, 2), jnp.float32)] * _GLIMPSES +
      [jax.ShapeDtypeStruct((batch, _GLIMPSES), jnp.float32),
       jax.ShapeDtypeStruct((batch, _GLIMPSES), jnp.float32),
       jax.ShapeDtypeStruct((batch, ncls), jnp.float32)])
  out_specs = [pl.BlockSpec((tb, s.shape[1]), lambda i: (i, 0))
               for s in out_shapes]

  outs = pl.pallas_call(
      functools.partial(_fused_kernel, tb=tb, std=_STD),
      out_shape=out_shapes,
      grid=grid,
      in_specs=in_specs,
      out_specs=out_specs,
      scratch_shapes=[
          pltpu.VMEM((tb * _BIG, _BIG), jnp.float32),
          pltpu.VMEM((tb, 2), jnp.int32),
          pltpu.SMEM((tb, 2), jnp.int32),
          pltpu.VMEM((tb, hg + hl), jnp.float32),
          pltpu.VMEM((tb, hg + hl + hid), jnp.float32),
          pltpu.SemaphoreType.DMA,
      ],
      compiler_params=pltpu.CompilerParams(
          dimension_semantics=("parallel",)),
  )(x2d, l_0, noise, *params)

  locs = list(outs[:_GLIMPSES])
  baselines, log_pis, log_probas = outs[_GLIMPSES:]
  return locs, baselines, log_pis, log_probas


# speedup vs baseline: 157.6616x; 1.1772x over previous
import functools
import math

import jax
import jax.numpy as jnp
from jax.experimental import pallas as pl
from jax.experimental.pallas import tpu as pltpu

_IMG = 64
_PS = 8
_SCALE = 2
_BIG = _PS * _SCALE
_GLIMPSES = 6
_STD = 0.25
_TB = 512
_U = 16


def _make_pool_sel():
  sel = jnp.zeros((8 * _BIG, 8 * _PS), jnp.float32)
  for r in range(_PS):
    for c in range(_PS):
      sel = sel.at[_BIG * r + 2 * c, _PS * r + c].set(1.0)
  return sel


def _fused_kernel(
    x_ref, l0_ref, noise_ref,
    w1_ref, b1_ref, w2_ref, b2_ref, w34_ref, b34_ref,
    wc_ref, bc_ref, whd_ref, bhd_ref, sel_ref,
    loc0, loc1, loc2, loc3, loc4, loc5,
    base_out, logpi_out, logp_out,
    q_ref, cv_ref, cs_ref, u_ref, gh_ref, sem,
    *, tb, std):
  f32 = jnp.float32
  relu = lambda v: jnp.maximum(v, 0.0)
  loc_outs = (loc0, loc1, loc2, loc3, loc4, loc5)

  hg = w1_ref.shape[1]
  hl = w2_ref.shape[1]
  hid = wc_ref.shape[1]
  half = _BIG // 2

  iota_r = jax.lax.broadcasted_iota(jnp.int32, (_BIG, _BIG), 0)
  iota_c = jax.lax.broadcasted_iota(jnp.int32, (_BIG, _BIG), 1)

  def gather_to_q():
    def chunk(k, carry):
      for u in range(_U):
        b = k * _U + u
        fx = cs_ref[b, 0]
        fy = cs_ref[b, 1]
        ys0 = fy - half
        xs0 = fx - half
        ya = jnp.minimum(jnp.maximum((ys0 >> 3) << 3, 0), _IMG - 32)
        off = ys0 - ya
        start = pl.multiple_of(b * _IMG + ya, 8)
        rows = pltpu.roll(x_ref[pl.ds(start, 32), :], (-off) % 32,
                          axis=0)[0:_BIG, :]
        patch = pltpu.roll(rows, (-xs0) % _IMG, axis=1)[:, 0:_BIG]
        valid = ((iota_r + ys0 >= 0) & (iota_r + ys0 < _IMG) &
                 (iota_c + xs0 >= 0) & (iota_c + xs0 < _IMG))
        q_ref[pl.ds(b * _BIG, _BIG), :] = jnp.where(valid, patch, 0.0)
      return carry

    jax.lax.fori_loop(0, tb // _U, chunk, 0)

  def assemble_glimpse():
    rows = [q_ref[i::_BIG, :] for i in range(_BIG)]
    center = jnp.concatenate(
        [rows[half // 2 + r][:, half // 2:half // 2 + _PS]
         for r in range(_PS)], axis=1)
    rsum = [rows[2 * r] + rows[2 * r + 1] for r in range(_PS)]
    usum = jnp.concatenate(
        [v + pltpu.roll(v, _BIG - 1, axis=1) for v in rsum], axis=1)
    pooled = 0.25 * jnp.dot(usum, sel_ref[...],
                            preferred_element_type=f32)
    return jnp.concatenate([center, pooled], axis=1)

  h_t = jnp.zeros((tb, hid), f32)
  l_t = l0_ref[...]
  for t in range(_GLIMPSES):
    cv_ref[...] = (0.5 * ((l_t + 1.0) * float(_IMG))).astype(jnp.int32)
    copy = pltpu.make_async_copy(cv_ref, cs_ref, sem)
    copy.start()
    copy.wait()
    gather_to_q()
    glimpse = assemble_glimpse()

    what_pre = (jnp.dot(glimpse, w1_ref[...], preferred_element_type=f32)
                + b1_ref[...])
    w2 = w2_ref[...]
    where_pre = (l_t[:, 0:1] * w2[0:1, :] +
                 l_t[:, 1:2] * w2[1:2, :] + b2_ref[...])

    u_ref[:, 0:hg] = relu(what_pre)
    u_ref[:, hg:hg + hl] = relu(where_pre)
    g_t = relu(jnp.dot(u_ref[...], w34_ref[...], preferred_element_type=f32)
               + b34_ref[...])

    gh_ref[:, 0:hg + hl] = g_t
    gh_ref[:, hg + hl:hg + hl + hid] = h_t
    h_t = relu(jnp.dot(gh_ref[...], wc_ref[...], preferred_element_type=f32)
               + bc_ref[...])

    heads = (jnp.dot(h_t, whd_ref[...], preferred_element_type=f32)
             + bhd_ref[...])
    mu = jnp.tanh(heads[:, 0:2])
    baseline = relu(heads[:, 2:3])

    l_t = jnp.tanh(mu + noise_ref[t])
    z = (l_t - mu) * (1.0 / std)
    log_norm = -math.log(std) - 0.5 * math.log(2.0 * math.pi)
    log_pi = jnp.sum(-0.5 * z * z + log_norm, axis=1, keepdims=True)

    loc_outs[t][...] = l_t
    base_out[:, t:t + 1] = baseline
    logpi_out[:, t:t + 1] = log_pi
    if t == _GLIMPSES - 1:
      logits = heads[:, 3:]
      m = jnp.max(logits, axis=1, keepdims=True)
      shifted = logits - m
      lse = jnp.log(jnp.sum(jnp.exp(shifted), axis=1, keepdims=True))
      logp_out[...] = shifted - lse


def kernel(x, l_0, noise, w1, b1, w2, b2, w34, b34,
           w_core, b_core, w_heads, b_heads):
  batch, chans, height, width = x.shape
  hid = w_core.shape[1]
  hg = w1.shape[1]
  hl = w2.shape[1]
  ncls = w_heads.shape[1] - 3
  nt = noise.shape[0]

  x2d = x.reshape(batch * height, width)
  tb = min(_TB, batch)
  grid = (batch // tb,)
  sel = _make_pool_sel()

  params = (w1, b1, w2, b2, w34, b34, w_core, b_core, w_heads, b_heads, sel)
  in_specs = [
      pl.BlockSpec((tb * height, width), lambda i: (i, 0)),
      pl.BlockSpec((tb, 2), lambda i: (i, 0)),
      pl.BlockSpec((nt, tb, 2), lambda i: (0, i, 0)),
  ]
  in_specs += [pl.BlockSpec(p.shape, lambda i: (0,) * p.ndim) for p in params]
  out_shapes = (
      [jax.ShapeDtypeStruct((batch, 2), jnp.float32)] * _GLIMPSES +
      [jax.ShapeDtypeStruct((batch, _GLIMPSES), jnp.float32),
       jax.ShapeDtypeStruct((batch, _GLIMPSES), jnp.float32),
       jax.ShapeDtypeStruct((batch, ncls), jnp.float32)])
  out_specs = [pl.BlockSpec((tb, s.shape[1]), lambda i: (i, 0))
               for s in out_shapes]

  outs = pl.pallas_call(
      functools.partial(_fused_kernel, tb=tb, std=_STD),
      out_shape=out_shapes,
      grid=grid,
      in_specs=in_specs,
      out_specs=out_specs,
      scratch_shapes=[
          pltpu.VMEM((tb * _BIG, _BIG), jnp.float32),
          pltpu.VMEM((tb, 2), jnp.int32),
          pltpu.SMEM((tb, 2), jnp.int32),
          pltpu.VMEM((tb, hg + hl), jnp.float32),
          pltpu.VMEM((tb, hg + hl + hid), jnp.float32),
          pltpu.SemaphoreType.DMA,
      ],
      compiler_params=pltpu.CompilerParams(
          dimension_semantics=("parallel",)),
  )(x2d, l_0, noise, *params)

  locs = list(outs[:_GLIMPSES])
  baselines, log_pis, log_probas = outs[_GLIMPSES:]
  return locs, baselines, log_pis, log_probas
